# baseline (speedup 1.0000x reference)
import functools

import jax
import jax.numpy as jnp
from jax import lax
from jax.experimental import pallas as pl
from jax.experimental.pallas import tpu as pltpu
from jax.experimental.pallas import tpu_sc as plsc

_B, _T, _Z = 4096, 65, 128
_NW = 32
_RPW = _B // _NW
_L = 16


def _rsqrt_nr(x):
    i = lax.bitcast_convert_type(x, jnp.int32)
    i = jnp.int32(0x5F3759DF) - (i >> 1)
    y = lax.bitcast_convert_type(i, jnp.float32)
    return y * (1.5 - 0.5 * x * y * y)


def _sc_body(dt_hbm, rt_hbm, zv_hbm, out_hbm,
             dt_v, rx_v, ry_v, rz_v, zv_v, out_v, sem):
    c = lax.axis_index("c")
    s = lax.axis_index("s")
    wid = s * 2 + c
    base = wid * _RPW
    sl = pl.ds(base, _RPW)
    cps = [pltpu.async_copy(dt_hbm.at[:, sl], dt_v, sem),
           pltpu.async_copy(rt_hbm.at[0, :, sl], rx_v, sem),
           pltpu.async_copy(rt_hbm.at[1, :, sl], ry_v, sem),
           pltpu.async_copy(rt_hbm.at[2, :, sl], rz_v, sem),
           pltpu.async_copy(zv_hbm.at[sl], zv_v, sem)]
    for cp in cps:
        cp.wait()

    nz = _Z // _L

    def ray(r, carry):
        r_s = jnp.full((_L,), r, jnp.int32)
        z = [zv_v[r, pl.ds(zi * _L, _L)] for zi in range(nz)]
        lo = [jnp.zeros((_L,), jnp.int32) for _ in range(nz)]
        for step in (32, 16, 8, 4, 2, 1):
            dp = [plsc.load_gather(dt_v, [lo[zi] + step, r_s])
                  for zi in range(nz)]
            for zi in range(nz):
                lo[zi] = jnp.where(dp[zi] <= z[zi], lo[zi] + step, lo[zi])
        for zi in range(nz):
            d0 = plsc.load_gather(dt_v, [lo[zi], r_s])
            vpos = z[zi] - d0
            hi = lo[zi] + 1
            c0 = [plsc.load_gather(pv, [lo[zi], r_s])
                  for pv in (rx_v, ry_v, rz_v)]
            c1 = [plsc.load_gather(pv, [hi, r_s])
                  for pv in (rx_v, ry_v, rz_v)]
            m = [c1[k] - c0[k] for k in range(3)]
            n2 = m[0] * m[0] + m[1] * m[1] + m[2] * m[2]
            scale = vpos * _rsqrt_nr(n2)
            for k in range(3):
                out_v[k, r, pl.ds(zi * _L, _L)] = c0[k] + scale * m[k]
        return carry

    lax.fori_loop(0, _RPW, ray, 0)
    for k in range(3):
        pltpu.sync_copy(out_v.at[k], out_hbm.at[k, sl])


@functools.partial(
    pl.kernel,
    out_type=jax.ShapeDtypeStruct((3, _B, _Z), jnp.float32),
    mesh=plsc.VectorSubcoreMesh(core_axis_name="c", subcore_axis_name="s"),
    compiler_params=pltpu.CompilerParams(needs_layout_passes=False),
    scratch_types=[
        pltpu.VMEM((_T, _RPW), jnp.float32),
        pltpu.VMEM((_T, _RPW), jnp.float32),
        pltpu.VMEM((_T, _RPW), jnp.float32),
        pltpu.VMEM((_T, _RPW), jnp.float32),
        pltpu.VMEM((_RPW, _Z), jnp.float32),
        pltpu.VMEM((3, _RPW, _Z), jnp.float32),
        pltpu.SemaphoreType.DMA,
    ],
)
def _evolution_sc(dt_hbm, rt_hbm, zv_hbm, out_hbm,
                  dt_v, rx_v, ry_v, rz_v, zv_v, out_v, sem):
    _sc_body(dt_hbm, rt_hbm, zv_hbm, out_hbm,
             dt_v, rx_v, ry_v, rz_v, zv_v, out_v, sem)


def kernel(r_hist, distances, z_vals):
    zv = z_vals.reshape(_B, _Z)
    dt = distances.T
    rt = r_hist.transpose(2, 1, 0)
    out = _evolution_sc(dt, rt, zv)
    return out.transpose(1, 2, 0)


# speedup vs baseline: 1.8073x; 1.8073x over previous
import functools

import jax
import jax.numpy as jnp
from jax import lax
from jax.experimental import pallas as pl
from jax.experimental.pallas import tpu as pltpu
from jax.experimental.pallas import tpu_sc as plsc

_B, _T, _Z = 4096, 65, 128
_NW = 32
_RPW = _B // _NW
_L = 16
_CH = 32
_NCH = _RPW // _CH


def _rsqrt_fast(x):
    i = lax.bitcast_convert_type(x, jnp.int32)
    i = jnp.int32(0x5F3759DF) - (i >> 1)
    y = lax.bitcast_convert_type(i, jnp.float32)
    return y * (1.5 - 0.5 * x * y * y)


def _sc_body(dist_hbm, rx_hbm, ry_hbm, rz_hbm, zv_hbm, out_hbm,
             dist_v, rx_v, ry_v, rz_v, zv_v, out_v,
             s0, s1, s2, s3, so):
    c = lax.axis_index("c")
    s = lax.axis_index("s")
    wid = s * 2 + c
    base = wid * _RPW
    sems = (s0, s1, s2, s3)
    in_cps = []
    for g in range(_NCH):
        hsl = pl.ds(base + g * _CH, _CH)
        vsl = pl.ds(g * _CH, _CH)
        in_cps.append([
            pltpu.async_copy(src.at[hsl], dst.at[vsl], sems[g])
            for src, dst in ((dist_hbm, dist_v), (rx_hbm, rx_v),
                             (ry_hbm, ry_v), (rz_hbm, rz_v), (zv_hbm, zv_v))])

    nz = _Z // _L

    def ray(r, carry):
        r_s = jnp.full((_L,), r, jnp.int32)
        z = [zv_v[r, pl.ds(zi * _L, _L)] for zi in range(nz)]
        lo = [jnp.zeros((_L,), jnp.int32) for _ in range(nz)]
        for step in (32, 16, 8, 4, 2, 1):
            dp = [plsc.load_gather(dist_v, [r_s, lo[zi] + step])
                  for zi in range(nz)]
            for zi in range(nz):
                lo[zi] = jnp.where(dp[zi] <= z[zi], lo[zi] + step, lo[zi])
        for zi in range(nz):
            d0 = plsc.load_gather(dist_v, [r_s, lo[zi]])
            vpos = z[zi] - d0
            hi = lo[zi] + 1
            c0 = [plsc.load_gather(pv, [r_s, lo[zi]])
                  for pv in (rx_v, ry_v, rz_v)]
            c1 = [plsc.load_gather(pv, [r_s, hi])
                  for pv in (rx_v, ry_v, rz_v)]
            m = [c1[k] - c0[k] for k in range(3)]
            n2 = m[0] * m[0] + m[1] * m[1] + m[2] * m[2]
            scale = vpos * _rsqrt_fast(n2)
            for k in range(3):
                out_v[k, r, pl.ds(zi * _L, _L)] = c0[k] + scale * m[k]
        return carry

    out_cps = []
    for g in range(_NCH):
        for cp in in_cps[g]:
            cp.wait()
        lax.fori_loop(g * _CH, (g + 1) * _CH, ray, 0)
        hsl = pl.ds(base + g * _CH, _CH)
        vsl = pl.ds(g * _CH, _CH)
        out_cps.extend(
            pltpu.async_copy(out_v.at[k, vsl], out_hbm.at[k, hsl], so)
            for k in range(3))
    for cp in out_cps:
        cp.wait()


@functools.partial(
    pl.kernel,
    out_type=jax.ShapeDtypeStruct((3, _B, _Z), jnp.float32),
    mesh=plsc.VectorSubcoreMesh(core_axis_name="c", subcore_axis_name="s"),
    compiler_params=pltpu.CompilerParams(needs_layout_passes=False),
    scratch_types=[
        pltpu.VMEM((_RPW, _T), jnp.float32),
        pltpu.VMEM((_RPW, _T), jnp.float32),
        pltpu.VMEM((_RPW, _T), jnp.float32),
        pltpu.VMEM((_RPW, _T), jnp.float32),
        pltpu.VMEM((_RPW, _Z), jnp.float32),
        pltpu.VMEM((3, _RPW, _Z), jnp.float32),
        pltpu.SemaphoreType.DMA,
        pltpu.SemaphoreType.DMA,
        pltpu.SemaphoreType.DMA,
        pltpu.SemaphoreType.DMA,
        pltpu.SemaphoreType.DMA,
    ],
)
def _evolution_sc(dist_hbm, rx_hbm, ry_hbm, rz_hbm, zv_hbm, out_hbm,
                  dist_v, rx_v, ry_v, rz_v, zv_v, out_v, s0, s1, s2, s3, so):
    _sc_body(dist_hbm, rx_hbm, ry_hbm, rz_hbm, zv_hbm, out_hbm,
             dist_v, rx_v, ry_v, rz_v, zv_v, out_v, s0, s1, s2, s3, so)


def kernel(r_hist, distances, z_vals):
    zv = z_vals.reshape(_B, _Z)
    rx = r_hist[:, :, 0]
    ry = r_hist[:, :, 1]
    rz = r_hist[:, :, 2]
    out = _evolution_sc(distances, rx, ry, rz, zv)
    return out.transpose(1, 2, 0)


# speedup vs baseline: 1.9645x; 1.0870x over previous
import functools

import jax
import jax.numpy as jnp
from jax import lax
from jax.experimental import pallas as pl
from jax.experimental.pallas import tpu as pltpu
from jax.experimental.pallas import tpu_sc as plsc

_B, _T, _Z = 4096, 65, 128
_NW = 32
_RPW = _B // _NW
_L = 16
_CH = 32
_NCH = _RPW // _CH


def _rsqrt_fast(x):
    i = lax.bitcast_convert_type(x, jnp.int32)
    i = jnp.int32(0x5F3759DF) - (i >> 1)
    return lax.bitcast_convert_type(i, jnp.float32)


def _sc_body(dist_hbm, rx_hbm, ry_hbm, rz_hbm, zv_hbm, out_hbm,
             dist_v, rx_v, ry_v, rz_v, zv_v, out_v,
             s0, s1, s2, s3, so):
    c = lax.axis_index("c")
    s = lax.axis_index("s")
    wid = s * 2 + c
    base = wid * _RPW
    sems = (s0, s1, s2, s3)
    in_cps = []
    for g in range(_NCH):
        hsl = pl.ds(base + g * _CH, _CH)
        vsl = pl.ds(g * _CH, _CH)
        in_cps.append([
            pltpu.async_copy(src.at[hsl], dst.at[vsl], sems[g])
            for src, dst in ((dist_hbm, dist_v), (rx_hbm, rx_v),
                             (ry_hbm, ry_v), (rz_hbm, rz_v), (zv_hbm, zv_v))])

    nz = _Z // _L

    def ray(r, carry):
        r_s = jnp.full((_L,), r, jnp.int32)
        z = [zv_v[r, pl.ds(zi * _L, _L)] for zi in range(nz)]
        lo = [jnp.zeros((_L,), jnp.int32) for _ in range(nz)]
        for step in (32, 16, 8, 4, 2, 1):
            dp = [plsc.load_gather(dist_v, [r_s, lo[zi] + step])
                  for zi in range(nz)]
            for zi in range(nz):
                lo[zi] = jnp.where(dp[zi] <= z[zi], lo[zi] + step, lo[zi])
        for zi in range(nz):
            d0 = plsc.load_gather(dist_v, [r_s, lo[zi]])
            vpos = z[zi] - d0
            hi = lo[zi] + 1
            c0 = [plsc.load_gather(pv, [r_s, lo[zi]])
                  for pv in (rx_v, ry_v, rz_v)]
            c1 = [plsc.load_gather(pv, [r_s, hi])
                  for pv in (rx_v, ry_v, rz_v)]
            m = [c1[k] - c0[k] for k in range(3)]
            n2 = m[0] * m[0] + m[1] * m[1] + m[2] * m[2]
            scale = vpos * _rsqrt_fast(n2)
            for k in range(3):
                out_v[k, r, pl.ds(zi * _L, _L)] = c0[k] + scale * m[k]
        return carry

    out_cps = []
    for g in range(_NCH):
        for cp in in_cps[g]:
            cp.wait()
        lax.fori_loop(g * _CH, (g + 1) * _CH, ray, 0)
        hsl = pl.ds(base + g * _CH, _CH)
        vsl = pl.ds(g * _CH, _CH)
        out_cps.extend(
            pltpu.async_copy(out_v.at[k, vsl], out_hbm.at[k, hsl], so)
            for k in range(3))
    for cp in out_cps:
        cp.wait()


@functools.partial(
    pl.kernel,
    out_type=jax.ShapeDtypeStruct((3, _B, _Z), jnp.float32),
    mesh=plsc.VectorSubcoreMesh(core_axis_name="c", subcore_axis_name="s"),
    compiler_params=pltpu.CompilerParams(needs_layout_passes=False),
    scratch_types=[
        pltpu.VMEM((_RPW, _T), jnp.float32),
        pltpu.VMEM((_RPW, _T), jnp.float32),
        pltpu.VMEM((_RPW, _T), jnp.float32),
        pltpu.VMEM((_RPW, _T), jnp.float32),
        pltpu.VMEM((_RPW, _Z), jnp.float32),
        pltpu.VMEM((3, _RPW, _Z), jnp.float32),
        pltpu.SemaphoreType.DMA,
        pltpu.SemaphoreType.DMA,
        pltpu.SemaphoreType.DMA,
        pltpu.SemaphoreType.DMA,
        pltpu.SemaphoreType.DMA,
    ],
)
def _evolution_sc(dist_hbm, rx_hbm, ry_hbm, rz_hbm, zv_hbm, out_hbm,
                  dist_v, rx_v, ry_v, rz_v, zv_v, out_v, s0, s1, s2, s3, so):
    _sc_body(dist_hbm, rx_hbm, ry_hbm, rz_hbm, zv_hbm, out_hbm,
             dist_v, rx_v, ry_v, rz_v, zv_v, out_v, s0, s1, s2, s3, so)


def kernel(r_hist, distances, z_vals):
    zv = z_vals.reshape(_B, _Z)
    rx = r_hist[:, :, 0]
    ry = r_hist[:, :, 1]
    rz = r_hist[:, :, 2]
    out = _evolution_sc(distances, rx, ry, rz, zv)
    return out.transpose(1, 2, 0)


# speedup vs baseline: 1.9921x; 1.0140x over previous
import functools

import jax
import jax.numpy as jnp
from jax import lax
from jax.experimental import pallas as pl
from jax.experimental.pallas import tpu as pltpu
from jax.experimental.pallas import tpu_sc as plsc

_B, _T, _Z = 4096, 65, 128
_NW = 32
_RPW = _B // _NW
_L = 16
_CH = 32
_NCH = _RPW // _CH


def _rsqrt_fast(x):
    i = lax.bitcast_convert_type(x, jnp.int32)
    i = jnp.int32(0x5F3759DF) - (i >> 1)
    return lax.bitcast_convert_type(i, jnp.float32)


def _sc_body(dist_hbm, rx_hbm, ry_hbm, rz_hbm, zv_hbm, out_hbm,
             dist_v, rx_v, ry_v, rz_v, zv_v, out_v,
             s0, s1, s2, s3, so):
    c = lax.axis_index("c")
    s = lax.axis_index("s")
    wid = s * 2 + c
    base = wid * _RPW
    sems = (s0, s1, s2, s3)
    in_cps = []
    for g in range(_NCH):
        hsl = pl.ds(base + g * _CH, _CH)
        vsl = pl.ds(g * _CH, _CH)
        in_cps.append([
            pltpu.async_copy(src.at[hsl], dst.at[vsl], sems[g])
            for src, dst in ((dist_hbm, dist_v), (rx_hbm, rx_v),
                             (ry_hbm, ry_v), (rz_hbm, rz_v), (zv_hbm, zv_v))])

    nz = _Z // _L

    k16 = jnp.full((_L,), 16, jnp.int32)
    k32 = jnp.full((_L,), 32, jnp.int32)
    k48 = jnp.full((_L,), 48, jnp.int32)

    def ray(r, carry):
        r_s = jnp.full((_L,), r, jnp.int32)
        z = [zv_v[r, pl.ds(zi * _L, _L)] for zi in range(nz)]
        d32 = plsc.load_gather(dist_v, [r_s, k32])
        d16 = plsc.load_gather(dist_v, [r_s, k16])
        d48 = plsc.load_gather(dist_v, [r_s, k48])
        lo = [None] * nz
        for zi in range(nz):
            m1 = d32 <= z[zi]
            l1 = jnp.where(m1, 32, 0)
            dv = jnp.where(m1, d48, d16)
            lo[zi] = l1 + jnp.where(dv <= z[zi], 16, 0)
        for step in (8, 4, 2, 1):
            dp = [plsc.load_gather(dist_v, [r_s, lo[zi] + step])
                  for zi in range(nz)]
            for zi in range(nz):
                lo[zi] = jnp.where(dp[zi] <= z[zi], lo[zi] + step, lo[zi])
        for zi in range(nz):
            d0 = plsc.load_gather(dist_v, [r_s, lo[zi]])
            vpos = z[zi] - d0
            hi = lo[zi] + 1
            c0 = [plsc.load_gather(pv, [r_s, lo[zi]])
                  for pv in (rx_v, ry_v, rz_v)]
            c1 = [plsc.load_gather(pv, [r_s, hi])
                  for pv in (rx_v, ry_v, rz_v)]
            m = [c1[k] - c0[k] for k in range(3)]
            n2 = m[0] * m[0] + m[1] * m[1] + m[2] * m[2]
            scale = vpos * _rsqrt_fast(n2)
            for k in range(3):
                out_v[k, r, pl.ds(zi * _L, _L)] = c0[k] + scale * m[k]
        return carry

    out_cps = []
    for g in range(_NCH):
        for cp in in_cps[g]:
            cp.wait()
        lax.fori_loop(g * _CH, (g + 1) * _CH, ray, 0)
        hsl = pl.ds(base + g * _CH, _CH)
        vsl = pl.ds(g * _CH, _CH)
        out_cps.extend(
            pltpu.async_copy(out_v.at[k, vsl], out_hbm.at[k, hsl], so)
            for k in range(3))
    for cp in out_cps:
        cp.wait()


@functools.partial(
    pl.kernel,
    out_type=jax.ShapeDtypeStruct((3, _B, _Z), jnp.float32),
    mesh=plsc.VectorSubcoreMesh(core_axis_name="c", subcore_axis_name="s"),
    compiler_params=pltpu.CompilerParams(needs_layout_passes=False),
    scratch_types=[
        pltpu.VMEM((_RPW, _T), jnp.float32),
        pltpu.VMEM((_RPW, _T), jnp.float32),
        pltpu.VMEM((_RPW, _T), jnp.float32),
        pltpu.VMEM((_RPW, _T), jnp.float32),
        pltpu.VMEM((_RPW, _Z), jnp.float32),
        pltpu.VMEM((3, _RPW, _Z), jnp.float32),
        pltpu.SemaphoreType.DMA,
        pltpu.SemaphoreType.DMA,
        pltpu.SemaphoreType.DMA,
        pltpu.SemaphoreType.DMA,
        pltpu.SemaphoreType.DMA,
    ],
)
def _evolution_sc(dist_hbm, rx_hbm, ry_hbm, rz_hbm, zv_hbm, out_hbm,
                  dist_v, rx_v, ry_v, rz_v, zv_v, out_v, s0, s1, s2, s3, so):
    _sc_body(dist_hbm, rx_hbm, ry_hbm, rz_hbm, zv_hbm, out_hbm,
             dist_v, rx_v, ry_v, rz_v, zv_v, out_v, s0, s1, s2, s3, so)


def kernel(r_hist, distances, z_vals):
    zv = z_vals.reshape(_B, _Z)
    rx = r_hist[:, :, 0]
    ry = r_hist[:, :, 1]
    rz = r_hist[:, :, 2]
    out = _evolution_sc(distances, rx, ry, rz, zv)
    return out.transpose(1, 2, 0)
